# baseline (speedup 1.0000x reference)
import functools

import jax
import jax.numpy as jnp
from jax import lax
from jax.experimental import pallas as pl
from jax.experimental.pallas import tpu as pltpu
from jax.experimental.pallas import tpu_sc as plsc

NC = 2
NS = 16
L = 16
EC = 256


def _ceil(a, b):
    return -(-a // b)


def _sc_degree(dstr, NP, K):
    stripe = NP // NS
    mesh = plsc.VectorSubcoreMesh(core_axis_name="c", subcore_axis_name="s")

    @functools.partial(
        pl.kernel,
        out_type=jax.ShapeDtypeStruct((NC, NP, L), jnp.float32),
        mesh=mesh,
        compiler_params=pltpu.CompilerParams(use_tc_tiling_on_sc=False),
        scratch_types=[
            pltpu.VMEM((K, EC), jnp.int32),
            pltpu.VMEM((EC, L), jnp.float32),
            pltpu.VMEM((EC, L), jnp.float32),
            pltpu.VMEM_SHARED((NP, L), jnp.float32),
        ],
    )
    def k(dst_hbm, out_hbm, dst_v, zero_v, one_v, acc):
        c = lax.axis_index("c")
        s = lax.axis_index("s")
        w = (1 - c) * NS + s

        def fill(i, _):
            zero_v[i, :] = jnp.zeros((L,), jnp.float32)
            one_v[i, :] = jnp.ones((L,), jnp.float32)
            return _

        lax.fori_loop(0, EC, fill, 0)
        tb = s * stripe
        for b in range(stripe // EC):
            pltpu.sync_copy(zero_v, acc.at[pl.ds(tb + b * EC, EC)])
        rem = stripe - (stripe // EC) * EC
        if rem:
            pltpu.sync_copy(zero_v.at[pl.ds(0, rem)],
                            acc.at[pl.ds(tb + (stripe // EC) * EC, rem)])
        pltpu.sync_copy(dst_hbm.at[pl.ds(w * K, K)], dst_v)
        plsc.subcore_barrier()

        def chunk(j, _):
            pltpu.sync_copy(one_v, acc.at[dst_v.at[j]], add=True)
            return _

        lax.fori_loop(0, K, chunk, 0)
        plsc.subcore_barrier()
        pltpu.sync_copy(acc.at[pl.ds(tb, stripe)],
                        out_hbm.at[c, pl.ds(tb, stripe)])

    return k(dstr)


def _sc_propagate(table, srcr, dstr, K):
    NP, D = table.shape
    stripe = NP // NS
    mesh = plsc.VectorSubcoreMesh(core_axis_name="c", subcore_axis_name="s")

    @functools.partial(
        pl.kernel,
        out_type=jax.ShapeDtypeStruct((NC, NP, D), jnp.float32),
        mesh=mesh,
        compiler_params=pltpu.CompilerParams(use_tc_tiling_on_sc=False),
        scratch_types=[
            pltpu.VMEM((K, EC), jnp.int32),
            pltpu.VMEM((K, EC), jnp.int32),
            pltpu.VMEM((EC, D), jnp.float32),
            pltpu.VMEM((EC, D), jnp.float32),
            pltpu.VMEM_SHARED((NP, D), jnp.float32),
            pltpu.SemaphoreType.DMA,
            pltpu.SemaphoreType.DMA,
        ],
    )
    def k(table_hbm, src_hbm, dst_hbm, out_hbm, src_v, dst_v, rows_a, rows_b,
          acc, sem_a, sem_b):
        c = lax.axis_index("c")
        s = lax.axis_index("s")
        w = (1 - c) * NS + s

        def zrow(i, _):
            for t in range(D // L):
                rows_a[i, pl.ds(t * L, L)] = jnp.zeros((L,), jnp.float32)
            return _

        lax.fori_loop(0, EC, zrow, 0)
        tb = s * stripe
        for b in range(stripe // EC):
            pltpu.sync_copy(rows_a, acc.at[pl.ds(tb + b * EC, EC)])
        rem = stripe - (stripe // EC) * EC
        if rem:
            pltpu.sync_copy(rows_a.at[pl.ds(0, rem)],
                            acc.at[pl.ds(tb + (stripe // EC) * EC, rem)])
        pltpu.sync_copy(src_hbm.at[pl.ds(w * K, K)], src_v)
        pltpu.sync_copy(dst_hbm.at[pl.ds(w * K, K)], dst_v)
        plsc.subcore_barrier()

        K2 = K // 2
        pltpu.async_copy(table_hbm.at[src_v.at[0]], rows_a, sem_a)

        def chunk2(jj, _):
            j = 2 * jj
            pltpu.make_async_copy(table_hbm.at[src_v.at[j]], rows_a,
                                  sem_a).wait()
            pltpu.async_copy(table_hbm.at[src_v.at[j + 1]], rows_b, sem_b)
            pltpu.sync_copy(rows_a, acc.at[dst_v.at[j]], add=True)
            pltpu.make_async_copy(table_hbm.at[src_v.at[j + 1]], rows_b,
                                  sem_b).wait()

            @pl.when(jj < K2 - 1)
            def _issue():
                pltpu.async_copy(table_hbm.at[src_v.at[j + 2]], rows_a, sem_a)

            pltpu.sync_copy(rows_b, acc.at[dst_v.at[j + 1]], add=True)
            return _

        lax.fori_loop(0, K2, chunk2, 0)
        plsc.subcore_barrier()
        pltpu.sync_copy(acc.at[pl.ds(tb, stripe)],
                        out_hbm.at[c, pl.ds(tb, stripe)])

    return k(table, srcr, dstr)


def _tc_matmul(xp, W1p):
    NP, D = xp.shape
    Hp = W1p.shape[1]
    RB = NP // 8

    def body(x_ref, w_ref, o_ref):
        o_ref[...] = jnp.dot(x_ref[...], w_ref[...],
                             preferred_element_type=jnp.float32)

    return pl.pallas_call(
        body,
        grid=(8,),
        in_specs=[pl.BlockSpec((RB, D), lambda i: (i, 0)),
                  pl.BlockSpec((D, Hp), lambda i: (0, 0))],
        out_specs=pl.BlockSpec((RB, Hp), lambda i: (i, 0)),
        out_shape=jax.ShapeDtypeStruct((NP, Hp), jnp.float32),
    )(xp, W1p)


def _tc_scale(h1, degp):
    NP, Hp = h1.shape
    RB = NP // 8

    def body(deg_ref, h1_ref, h1s_ref, dinv_ref):
        deg = deg_ref[0] + deg_ref[1] + 1.0
        dinv = lax.rsqrt(jnp.maximum(deg, 1.0))
        dinv_ref[...] = dinv
        h1s_ref[...] = h1_ref[...] * dinv[:, 0:1]

    return pl.pallas_call(
        body,
        grid=(8,),
        in_specs=[pl.BlockSpec((NC, RB, L), lambda i: (0, i, 0)),
                  pl.BlockSpec((RB, Hp), lambda i: (i, 0))],
        out_specs=[pl.BlockSpec((RB, Hp), lambda i: (i, 0)),
                   pl.BlockSpec((RB, L), lambda i: (i, 0))],
        out_shape=[jax.ShapeDtypeStruct((NP, Hp), jnp.float32),
                   jax.ShapeDtypeStruct((NP, L), jnp.float32)],
    )(degp, h1)


def _tc_layer2(p, h1s, dinv, b1p, W2p):
    NP, Hp = h1s.shape
    Cp = W2p.shape[1]
    RB = NP // 8

    def body(p_ref, h1s_ref, dinv_ref, b1_ref, w2_ref, gs_ref):
        d = dinv_ref[:, 0:1]
        h = jnp.maximum((p_ref[0] + p_ref[1] + h1s_ref[...]) * d + b1_ref[...],
                        0.0)
        gs_ref[...] = jnp.dot(h, w2_ref[...],
                              preferred_element_type=jnp.float32) * d

    return pl.pallas_call(
        body,
        grid=(8,),
        in_specs=[pl.BlockSpec((NC, RB, Hp), lambda i: (0, i, 0)),
                  pl.BlockSpec((RB, Hp), lambda i: (i, 0)),
                  pl.BlockSpec((RB, L), lambda i: (i, 0)),
                  pl.BlockSpec((1, Hp), lambda i: (0, 0)),
                  pl.BlockSpec((Hp, Cp), lambda i: (0, 0))],
        out_specs=pl.BlockSpec((RB, Cp), lambda i: (i, 0)),
        out_shape=jax.ShapeDtypeStruct((NP, Cp), jnp.float32),
    )(p, h1s, dinv, b1p, W2p)


def _tc_final(q, gs, dinv, b2p, Wcp, bcp):
    NP, Cp = gs.shape
    RB = NP // 8

    def body(q_ref, gs_ref, dinv_ref, b2_ref, wc_ref, bc_ref, h2_ref, out_ref):
        d = dinv_ref[:, 0:1]
        h2 = (q_ref[0] + q_ref[1] + gs_ref[...]) * d + b2_ref[...]
        h2_ref[...] = h2
        out_ref[...] = jnp.dot(h2, wc_ref[...],
                               preferred_element_type=jnp.float32) + bc_ref[...]

    return pl.pallas_call(
        body,
        grid=(8,),
        in_specs=[pl.BlockSpec((NC, RB, Cp), lambda i: (0, i, 0)),
                  pl.BlockSpec((RB, Cp), lambda i: (i, 0)),
                  pl.BlockSpec((RB, L), lambda i: (i, 0)),
                  pl.BlockSpec((1, Cp), lambda i: (0, 0)),
                  pl.BlockSpec((Cp, Cp), lambda i: (0, 0)),
                  pl.BlockSpec((1, Cp), lambda i: (0, 0))],
        out_specs=[pl.BlockSpec((RB, Cp), lambda i: (i, 0)),
                   pl.BlockSpec((RB, Cp), lambda i: (i, 0))],
        out_shape=[jax.ShapeDtypeStruct((NP, Cp), jnp.float32),
                   jax.ShapeDtypeStruct((NP, Cp), jnp.float32)],
    )(q, gs, dinv, b2p, Wcp, bcp)


def kernel(x, edge_index, W1, b1, W2, b2, Wc, bc):
    N, D = x.shape
    H = W1.shape[1]
    C = W2.shape[1]
    E = edge_index.shape[1]
    NW = NC * NS
    NP = _ceil(N + 1, 128) * 128
    Hp = _ceil(H, L) * L
    Cp = L
    K = _ceil(_ceil(E, NW * EC), 8) * 8
    Ep = K * NW * EC

    pad = jnp.full((Ep - E,), N, dtype=jnp.int32)
    srcr = jnp.concatenate([edge_index[0].astype(jnp.int32), pad]).reshape(NW * K, EC)
    dstr = jnp.concatenate([edge_index[1].astype(jnp.int32), pad]).reshape(NW * K, EC)

    xp = jnp.pad(x, ((0, NP - N), (0, 0)))
    W1p = jnp.pad(W1, ((0, 0), (0, Hp - H)))
    b1p = jnp.pad(b1, (0, Hp - H))[None, :]
    W2p = jnp.pad(W2, ((0, Hp - H), (0, Cp - C)))
    b2p = jnp.pad(b2, (0, Cp - C))[None, :]
    Wcp = jnp.pad(Wc, ((0, Cp - C), (0, Cp - C)))
    bcp = jnp.pad(bc, (0, Cp - C))[None, :]

    degp = _sc_degree(dstr, NP, K)
    h1 = _tc_matmul(xp, W1p)
    h1s, dinv = _tc_scale(h1, degp)
    p = _sc_propagate(h1s, srcr, dstr, K)
    gs = _tc_layer2(p, h1s, dinv, b1p, W2p)
    q = _sc_propagate(gs, srcr, dstr, K)
    h2p, outp = _tc_final(q, gs, dinv, b2p, Wcp, bcp)
    return (outp[:N, :C], h2p[:N, :C])


# speedup vs baseline: 1.9907x; 1.9907x over previous
import functools

import jax
import jax.numpy as jnp
from jax import lax
from jax.experimental import pallas as pl
from jax.experimental.pallas import tpu as pltpu
from jax.experimental.pallas import tpu_sc as plsc

NC = 2
NS = 16
L = 16
EC = 256


def _ceil(a, b):
    return -(-a // b)


def _sc_degree(dstr, NP, K):
    stripe = NP // NS
    mesh = plsc.VectorSubcoreMesh(core_axis_name="c", subcore_axis_name="s")

    @functools.partial(
        pl.kernel,
        out_type=jax.ShapeDtypeStruct((NC, NP, L), jnp.float32),
        mesh=mesh,
        compiler_params=pltpu.CompilerParams(use_tc_tiling_on_sc=False),
        scratch_types=[
            pltpu.VMEM((K, EC), jnp.int32),
            pltpu.VMEM((EC, L), jnp.float32),
            pltpu.VMEM((EC, L), jnp.float32),
            pltpu.VMEM_SHARED((NP, L), jnp.float32),
        ],
    )
    def k(dst_hbm, out_hbm, dst_v, zero_v, one_v, acc):
        c = lax.axis_index("c")
        s = lax.axis_index("s")
        w = c * NS + s

        def fill(i, _):
            zero_v[i, :] = jnp.zeros((L,), jnp.float32)
            one_v[i, :] = jnp.ones((L,), jnp.float32)
            return _

        lax.fori_loop(0, EC, fill, 0)
        tb = s * stripe
        for b in range(stripe // EC):
            pltpu.sync_copy(zero_v, acc.at[pl.ds(tb + b * EC, EC)])
        rem = stripe - (stripe // EC) * EC
        if rem:
            pltpu.sync_copy(zero_v.at[pl.ds(0, rem)],
                            acc.at[pl.ds(tb + (stripe // EC) * EC, rem)])
        pltpu.sync_copy(dst_hbm.at[pl.ds(w * K, K)], dst_v)
        plsc.subcore_barrier()

        def chunk(j, _):
            pltpu.sync_copy(one_v, acc.at[dst_v.at[j]], add=True)
            return _

        lax.fori_loop(0, K, chunk, 0)
        plsc.subcore_barrier()
        pltpu.sync_copy(acc.at[pl.ds(tb, stripe)],
                        out_hbm.at[c, pl.ds(tb, stripe)])

    return k(dstr)


def _sc_propagate(table, srcr, dstr, K):
    NP, D = table.shape
    stripe = NP // NS
    mesh = plsc.VectorSubcoreMesh(core_axis_name="c", subcore_axis_name="s")

    @functools.partial(
        pl.kernel,
        out_type=jax.ShapeDtypeStruct((NC, NP, D), jnp.float32),
        mesh=mesh,
        compiler_params=pltpu.CompilerParams(use_tc_tiling_on_sc=False),
        scratch_types=[
            pltpu.VMEM((K, EC), jnp.int32),
            pltpu.VMEM((K, EC), jnp.int32),
            pltpu.VMEM((EC, D), jnp.float32),
            pltpu.VMEM((EC, D), jnp.float32),
            pltpu.VMEM_SHARED((NP, D), jnp.float32),
            pltpu.SemaphoreType.DMA,
            pltpu.SemaphoreType.DMA,
        ],
    )
    def k(table_hbm, src_hbm, dst_hbm, out_hbm, src_v, dst_v, rows_a, rows_b,
          acc, sem_a, sem_b):
        c = lax.axis_index("c")
        s = lax.axis_index("s")
        w = c * NS + s

        def zrow(i, _):
            for t in range(D // L):
                rows_a[i, pl.ds(t * L, L)] = jnp.zeros((L,), jnp.float32)
            return _

        lax.fori_loop(0, EC, zrow, 0)
        tb = s * stripe
        for b in range(stripe // EC):
            pltpu.sync_copy(rows_a, acc.at[pl.ds(tb + b * EC, EC)])
        rem = stripe - (stripe // EC) * EC
        if rem:
            pltpu.sync_copy(rows_a.at[pl.ds(0, rem)],
                            acc.at[pl.ds(tb + (stripe // EC) * EC, rem)])
        pltpu.sync_copy(src_hbm.at[pl.ds(w * K, K)], src_v)
        pltpu.sync_copy(dst_hbm.at[pl.ds(w * K, K)], dst_v)
        plsc.subcore_barrier()

        K2 = K // 2
        pltpu.async_copy(table_hbm.at[src_v.at[0]], rows_a, sem_a)

        def chunk2(jj, _):
            j = 2 * jj
            pltpu.make_async_copy(table_hbm.at[src_v.at[j]], rows_a,
                                  sem_a).wait()
            pltpu.async_copy(table_hbm.at[src_v.at[j + 1]], rows_b, sem_b)
            pltpu.sync_copy(rows_a, acc.at[dst_v.at[j]], add=True)
            pltpu.make_async_copy(table_hbm.at[src_v.at[j + 1]], rows_b,
                                  sem_b).wait()

            @pl.when(jj < K2 - 1)
            def _issue():
                pltpu.async_copy(table_hbm.at[src_v.at[j + 2]], rows_a, sem_a)

            pltpu.sync_copy(rows_b, acc.at[dst_v.at[j + 1]], add=True)
            return _

        lax.fori_loop(0, K2, chunk2, 0)
        plsc.subcore_barrier()
        pltpu.sync_copy(acc.at[pl.ds(tb, stripe)],
                        out_hbm.at[c, pl.ds(tb, stripe)])

    return k(table, srcr, dstr)


def _tc_matmul(xp, W1p):
    NP, D = xp.shape
    Hp = W1p.shape[1]
    RB = NP // 8

    def body(x_ref, w_ref, o_ref):
        o_ref[...] = jnp.dot(x_ref[...], w_ref[...],
                             preferred_element_type=jnp.float32)

    return pl.pallas_call(
        body,
        grid=(8,),
        in_specs=[pl.BlockSpec((RB, D), lambda i: (i, 0)),
                  pl.BlockSpec((D, Hp), lambda i: (0, 0))],
        out_specs=pl.BlockSpec((RB, Hp), lambda i: (i, 0)),
        out_shape=jax.ShapeDtypeStruct((NP, Hp), jnp.float32),
    )(xp, W1p)


def _tc_scale(h1, degp):
    NP, Hp = h1.shape
    RB = NP // 8

    def body(deg_ref, h1_ref, h1s_ref, dinv_ref):
        deg = deg_ref[0] + deg_ref[1] + 1.0
        dinv = lax.rsqrt(jnp.maximum(deg, 1.0))
        dinv_ref[...] = dinv
        h1s_ref[...] = h1_ref[...] * dinv[:, 0:1]

    return pl.pallas_call(
        body,
        grid=(8,),
        in_specs=[pl.BlockSpec((NC, RB, L), lambda i: (0, i, 0)),
                  pl.BlockSpec((RB, Hp), lambda i: (i, 0))],
        out_specs=[pl.BlockSpec((RB, Hp), lambda i: (i, 0)),
                   pl.BlockSpec((RB, L), lambda i: (i, 0))],
        out_shape=[jax.ShapeDtypeStruct((NP, Hp), jnp.float32),
                   jax.ShapeDtypeStruct((NP, L), jnp.float32)],
    )(degp, h1)


def _tc_layer2(p, h1s, dinv, b1p, W2p):
    NP, Hp = h1s.shape
    Cp = W2p.shape[1]
    RB = NP // 8

    def body(p_ref, h1s_ref, dinv_ref, b1_ref, w2_ref, gs_ref):
        d = dinv_ref[:, 0:1]
        h = jnp.maximum((p_ref[0] + p_ref[1] + h1s_ref[...]) * d + b1_ref[...],
                        0.0)
        gs_ref[...] = jnp.dot(h, w2_ref[...],
                              preferred_element_type=jnp.float32) * d

    return pl.pallas_call(
        body,
        grid=(8,),
        in_specs=[pl.BlockSpec((NC, RB, Hp), lambda i: (0, i, 0)),
                  pl.BlockSpec((RB, Hp), lambda i: (i, 0)),
                  pl.BlockSpec((RB, L), lambda i: (i, 0)),
                  pl.BlockSpec((1, Hp), lambda i: (0, 0)),
                  pl.BlockSpec((Hp, Cp), lambda i: (0, 0))],
        out_specs=pl.BlockSpec((RB, Cp), lambda i: (i, 0)),
        out_shape=jax.ShapeDtypeStruct((NP, Cp), jnp.float32),
    )(p, h1s, dinv, b1p, W2p)


def _tc_final(q, gs, dinv, b2p, Wcp, bcp):
    NP, Cp = gs.shape
    RB = NP // 8

    def body(q_ref, gs_ref, dinv_ref, b2_ref, wc_ref, bc_ref, h2_ref, out_ref):
        d = dinv_ref[:, 0:1]
        h2 = (q_ref[0] + q_ref[1] + gs_ref[...]) * d + b2_ref[...]
        h2_ref[...] = h2
        out_ref[...] = jnp.dot(h2, wc_ref[...],
                               preferred_element_type=jnp.float32) + bc_ref[...]

    return pl.pallas_call(
        body,
        grid=(8,),
        in_specs=[pl.BlockSpec((NC, RB, Cp), lambda i: (0, i, 0)),
                  pl.BlockSpec((RB, Cp), lambda i: (i, 0)),
                  pl.BlockSpec((RB, L), lambda i: (i, 0)),
                  pl.BlockSpec((1, Cp), lambda i: (0, 0)),
                  pl.BlockSpec((Cp, Cp), lambda i: (0, 0)),
                  pl.BlockSpec((1, Cp), lambda i: (0, 0))],
        out_specs=[pl.BlockSpec((RB, Cp), lambda i: (i, 0)),
                   pl.BlockSpec((RB, Cp), lambda i: (i, 0))],
        out_shape=[jax.ShapeDtypeStruct((NP, Cp), jnp.float32),
                   jax.ShapeDtypeStruct((NP, Cp), jnp.float32)],
    )(q, gs, dinv, b2p, Wcp, bcp)


def kernel(x, edge_index, W1, b1, W2, b2, Wc, bc):
    N, D = x.shape
    H = W1.shape[1]
    C = W2.shape[1]
    E = edge_index.shape[1]
    NW = NC * NS
    NP = _ceil(N + 1, 128) * 128
    Hp = _ceil(H, L) * L
    Cp = L
    K = _ceil(_ceil(E, NW * EC), 8) * 8
    Ep = K * NW * EC

    pad = N + (jnp.arange(Ep - E, dtype=jnp.int32) % (NP - N))
    srcr = jnp.concatenate([edge_index[0].astype(jnp.int32), pad]).reshape(NW * K, EC)
    dstr = jnp.concatenate([edge_index[1].astype(jnp.int32), pad]).reshape(NW * K, EC)

    xp = jnp.pad(x, ((0, NP - N), (0, 0)))
    W1p = jnp.pad(W1, ((0, 0), (0, Hp - H)))
    b1p = jnp.pad(b1, (0, Hp - H))[None, :]
    W2p = jnp.pad(W2, ((0, Hp - H), (0, Cp - C)))
    b2p = jnp.pad(b2, (0, Cp - C))[None, :]
    Wcp = jnp.pad(Wc, ((0, Cp - C), (0, Cp - C)))
    bcp = jnp.pad(bc, (0, Cp - C))[None, :]

    degp = _sc_degree(dstr, NP, K)
    h1 = _tc_matmul(xp, W1p)
    h1s, dinv = _tc_scale(h1, degp)
    p = _sc_propagate(h1s, srcr, dstr, K)
    gs = _tc_layer2(p, h1s, dinv, b1p, W2p)
    q = _sc_propagate(gs, srcr, dstr, K)
    h2p, outp = _tc_final(q, gs, dinv, b2p, Wcp, bcp)
    return (outp[:N, :C], h2p[:N, :C])
